# Initial kernel scaffold; baseline (speedup 1.0000x reference)
import jax
import jax.numpy as jnp
from jax.experimental import pallas as pl


def kernel(x):
    raise NotImplementedError("write your pallas kernel here")


# speedup vs baseline: 17.7992x; 17.7992x over previous
import jax
import jax.numpy as jnp
from jax import lax
from jax.experimental import pallas as pl
from jax.experimental.pallas import tpu as pltpu
from jax.experimental.pallas import tpu_sc as plsc

W = 8
K = 16
F = 224
T = 224
FO = F - W + 1
BC = 4 * 192
N = FO * T
NEG = float("-inf")
GCH = 4

NC = 2
NS = 16
NW = NC * NS
SPW = BC // NW
LANES = 16
CH = 14
NCHUNK = N // (LANES * CH)


def _pool_body(x_ref, o_ref):
    a = x_ref[...]
    v = a[:, 0:FO, :]
    for d in range(1, W):
        v = v + a[:, d:d + FO, :]
    h = v
    for d in range(1, W):
        h = h + jnp.concatenate([v[:, :, d:], v[:, :, :d]], axis=2)
    col = lax.broadcasted_iota(jnp.int32, h.shape, 2)
    o_ref[...] = jnp.where(col < FO, h * (1.0 / (W * W)), NEG)


def _tc_pool(xr):
    n = xr.shape[0]
    return pl.pallas_call(
        _pool_body,
        grid=(n // GCH,),
        in_specs=[pl.BlockSpec((GCH, F, T), lambda i: (i, 0, 0))],
        out_specs=pl.BlockSpec((GCH, FO, T), lambda i: (i, 0, 0)),
        out_shape=jax.ShapeDtypeStruct((n, FO, T), jnp.float32),
    )(xr)


def _scan_slice(buf):
    def chunk(ci, carry):
        S, smin = carry
        off = ci * (CH * LANES)
        vs = [buf[pl.ds(off + j * LANES, LANES)] for j in range(CH)]
        m = vs[0]
        for v in vs[1:]:
            m = jnp.maximum(m, v)

        def dohit(c):
            S, smin = c
            for v in vs:
                def merge(c2, v=v):
                    S2, _ = c2
                    S3 = jnp.sort(jnp.maximum(S2, jnp.flip(jnp.sort(v), axis=0)))
                    return S3, jnp.full((LANES,), jnp.min(S3), jnp.float32)
                S, smin = lax.cond(jnp.any(v > smin), merge,
                                   lambda c2: c2, (S, smin))
            return S, smin

        return lax.cond(jnp.any(m > smin), dohit, lambda c: c, (S, smin))

    S0 = jnp.full((LANES,), NEG, jnp.float32)
    S, _ = lax.fori_loop(0, NCHUNK, chunk, (S0, S0))
    return S


def _sc_topk_body(flat, out, buf, res, sem):
    w = lax.axis_index("s") * NC + lax.axis_index("c")
    base = w * SPW
    zero = jnp.zeros((LANES,), jnp.float32)
    lanes = lax.broadcasted_iota(jnp.int32, (LANES,), 0)

    def slice_loop(i, carry):
        r0, r1 = carry
        pltpu.async_copy(flat.at[base + i], buf, sem).wait()
        S = _scan_slice(buf)
        t = jnp.sum(S) * (1.0 / K)
        r0 = jnp.where(lanes == i, t, r0)
        r1 = jnp.where(lanes == i - LANES, t, r1)
        return r0, r1

    r0, r1 = pl.loop(0, SPW, init_carry=(zero, zero))(slice_loop)
    res[pl.ds(0, LANES)] = r0
    res[pl.ds(LANES, LANES)] = r1
    pltpu.sync_copy(res.at[pl.ds(0, SPW)], out.at[pl.ds(base, SPW)])


_sc_topk = pl.kernel(
    _sc_topk_body,
    out_type=jax.ShapeDtypeStruct((BC,), jnp.float32),
    mesh=plsc.VectorSubcoreMesh(core_axis_name="c", subcore_axis_name="s",
                                num_cores=NC, num_subcores=NS),
    scratch_types=[
        pltpu.VMEM((N,), jnp.float32),
        pltpu.VMEM((2 * LANES,), jnp.float32),
        pltpu.SemaphoreType.DMA,
    ],
    compiler_params=pltpu.CompilerParams(needs_layout_passes=False),
)


def kernel(x):
    B, C, _, _ = x.shape
    xr = x.reshape(B * C, F, T)
    pooled = _tc_pool(xr)
    flat = pooled.reshape(B * C, N)
    out = _sc_topk(flat)
    return out.reshape(B, C)


# speedup vs baseline: 28.7911x; 1.6175x over previous
import jax
import jax.numpy as jnp
from jax import lax
from jax.experimental import pallas as pl
from jax.experimental.pallas import tpu as pltpu
from jax.experimental.pallas import tpu_sc as plsc

W = 8
K = 16
F = 224
T = 224
FO = F - W + 1
BC = 4 * 192
N = FO * T
NEG = float("-inf")
GCH = 4

NC = 2
NS = 16
NW = NC * NS
SPW = BC // NW
LANES = 16
NG = T // LANES


def _pool_body(x_ref, o_ref, rm_ref):
    a = x_ref[...]
    v = a[:, 0:FO, :]
    for d in range(1, W):
        v = v + a[:, d:d + FO, :]
    h = v
    for d in range(1, W):
        h = h + jnp.concatenate([v[:, :, d:], v[:, :, :d]], axis=2)
    col = lax.broadcasted_iota(jnp.int32, h.shape, 2)
    o = jnp.where(col < FO, h * (1.0 / (W * W)), NEG)
    o_ref[...] = o
    rm = jnp.max(o, axis=2)
    rm_ref[...] = jnp.concatenate(
        [rm, jnp.full((GCH, T - FO), NEG, jnp.float32)], axis=1)[:, None, :]


def _tc_pool(xr):
    n = xr.shape[0]
    return pl.pallas_call(
        _pool_body,
        grid=(n // GCH,),
        in_specs=[pl.BlockSpec((GCH, F, T), lambda i: (i, 0, 0))],
        out_specs=[pl.BlockSpec((GCH, FO, T), lambda i: (i, 0, 0)),
                   pl.BlockSpec((GCH, 1, T), lambda i: (i, 0, 0))],
        out_shape=[jax.ShapeDtypeStruct((n, FO, T), jnp.float32),
                   jax.ShapeDtypeStruct((n, 1, T), jnp.float32)],
    )(xr)


def _bmerge(a, b):
    return jnp.sort(jnp.maximum(a, jnp.flip(b)))


def _tournament(vs):
    level = [jnp.sort(v) for v in vs]
    while len(level) > 1:
        nxt = [_bmerge(level[i], level[i + 1])
               for i in range(0, len(level) - 1, 2)]
        if len(level) % 2:
            nxt.append(level[-1])
        level = nxt
    return level[0]


def _process_slice(buf, rmb):
    lanes = lax.broadcasted_iota(jnp.int32, (LANES,), 0)
    srm = _tournament([rmb[pl.ds(g * LANES, LANES)] for g in range(NG)])
    smin = jnp.full((LANES,), jnp.min(srm), jnp.float32)
    s0 = jnp.full((LANES,), NEG, jnp.float32)

    def group(g, carry):
        S, smin = carry
        rm = rmb[pl.ds(g * LANES, LANES)]
        mask = rm >= smin

        def cond(c):
            m, _, _ = c
            return jnp.any(m)

        def step(c):
            m, S, smin = c
            ffsv = plsc.all_reduce_ffs(m)
            row = (g * LANES + ffsv) * T
            vs = [plsc.load_gather(buf, [row + j * LANES + lanes])
                  for j in range(NG)]
            S = _bmerge(S, _tournament(vs))
            smin = jnp.full((LANES,), jnp.min(S), jnp.float32)
            m = jnp.logical_and(m, lanes != ffsv)
            m = jnp.logical_and(m, rm >= smin)
            return m, S, smin

        _, S, smin = lax.while_loop(cond, step, (mask, S, smin))
        return S, smin

    S, _ = lax.fori_loop(0, NG, group, (s0, smin))
    return jnp.sum(S) * (1.0 / K)


def _sc_topk_body(flat, rowmax, out, buf0, buf1, rmb0, rmb1, res,
                  sem0, sem1, semr0, semr1):
    w = lax.axis_index("s") * NC + lax.axis_index("c")
    base = w * SPW
    bufs, rmbs = (buf0, buf1), (rmb0, rmb1)
    sems, semrs = (sem0, sem1), (semr0, semr1)
    zero = jnp.zeros((LANES,), jnp.float32)
    lanes = lax.broadcasted_iota(jnp.int32, (LANES,), 0)

    for b in (0, 1):
        pltpu.async_copy(flat.at[base + b], bufs[b], sems[b])
        pltpu.async_copy(rowmax.at[base + b], rmbs[b], semrs[b])

    def pair(i, carry):
        r0, r1 = carry
        for b in (0, 1):
            sl = i + b
            pltpu.make_async_copy(flat.at[base], bufs[b], sems[b]).wait()
            pltpu.make_async_copy(rowmax.at[base], rmbs[b], semrs[b]).wait()
            t = _process_slice(bufs[b], rmbs[b])

            @pl.when(sl + 2 < SPW)
            def _():
                pltpu.async_copy(flat.at[base + sl + 2], bufs[b], sems[b])
                pltpu.async_copy(rowmax.at[base + sl + 2], rmbs[b], semrs[b])

            r0 = jnp.where(lanes == sl, t, r0)
            r1 = jnp.where(lanes == sl - LANES, t, r1)
        return r0, r1

    r0, r1 = pl.loop(0, SPW, step=2, init_carry=(zero, zero))(pair)
    res[pl.ds(0, LANES)] = r0
    res[pl.ds(LANES, LANES)] = r1
    pltpu.sync_copy(res.at[pl.ds(0, SPW)], out.at[pl.ds(base, SPW)])


_sc_topk = pl.kernel(
    _sc_topk_body,
    out_type=jax.ShapeDtypeStruct((BC,), jnp.float32),
    mesh=plsc.VectorSubcoreMesh(core_axis_name="c", subcore_axis_name="s",
                                num_cores=NC, num_subcores=NS),
    scratch_types=[
        pltpu.VMEM((N,), jnp.float32),
        pltpu.VMEM((N,), jnp.float32),
        pltpu.VMEM((T,), jnp.float32),
        pltpu.VMEM((T,), jnp.float32),
        pltpu.VMEM((2 * LANES,), jnp.float32),
        pltpu.SemaphoreType.DMA,
        pltpu.SemaphoreType.DMA,
        pltpu.SemaphoreType.DMA,
        pltpu.SemaphoreType.DMA,
    ],
    compiler_params=pltpu.CompilerParams(needs_layout_passes=False),
)


def kernel(x):
    B, C, _, _ = x.shape
    xr = x.reshape(B * C, F, T)
    pooled, rowmax = _tc_pool(xr)
    flat = pooled.reshape(B * C, N)
    out = _sc_topk(flat, rowmax.reshape(B * C, T))
    return out.reshape(B, C)


# speedup vs baseline: 45.2404x; 1.5713x over previous
import jax
import jax.numpy as jnp
from jax import lax
from jax.experimental import pallas as pl
from jax.experimental.pallas import tpu as pltpu
from jax.experimental.pallas import tpu_sc as plsc

W = 8
K = 16
F = 224
T = 224
FO = F - W + 1
BC = 4 * 192
N = FO * T
NEG = float("-inf")
GCH = 4

NC = 2
NS = 16
NW = NC * NS
SPW = BC // NW
LANES = 16
NG = T // LANES


def _pool_body(x_ref, o_ref, rm_ref):
    a = x_ref[...]
    b = a[:, :-1, :] + a[:, 1:, :]
    b = b[:, :-2, :] + b[:, 2:, :]
    v = b[:, :-4, :] + b[:, 4:, :]
    h = v
    for d in (1, 2, 4):
        h = h + jnp.concatenate([h[:, :, d:], h[:, :, :d]], axis=2)
    col = lax.broadcasted_iota(jnp.int32, h.shape, 2)
    o = jnp.where(col < FO, h * (1.0 / (W * W)), NEG)
    o_ref[...] = o
    rm = jnp.max(o, axis=2)
    rm_ref[...] = jnp.concatenate(
        [rm, jnp.full((GCH, T - FO), NEG, jnp.float32)], axis=1)[:, None, :]


def _tc_pool(xr):
    n = xr.shape[0]
    return pl.pallas_call(
        _pool_body,
        grid=(n // GCH,),
        in_specs=[pl.BlockSpec((GCH, F, T), lambda i: (i, 0, 0))],
        out_specs=[pl.BlockSpec((GCH, FO, T), lambda i: (i, 0, 0)),
                   pl.BlockSpec((GCH, 1, T), lambda i: (i, 0, 0))],
        out_shape=[jax.ShapeDtypeStruct((n, FO, T), jnp.float32),
                   jax.ShapeDtypeStruct((n, 1, T), jnp.float32)],
    )(xr)


def _bmerge(a, b):
    return jnp.sort(jnp.maximum(a, jnp.flip(b)))


def _tournament(vs):
    level = [jnp.sort(v) for v in vs]
    while len(level) > 1:
        nxt = [_bmerge(level[i], level[i + 1])
               for i in range(0, len(level) - 1, 2)]
        if len(level) % 2:
            nxt.append(level[-1])
        level = nxt
    return level[0]


def _process_slice(buf, rmb):
    lanes = lax.broadcasted_iota(jnp.int32, (LANES,), 0)
    srm = _tournament([rmb[pl.ds(g * LANES, LANES)] for g in range(NG)])
    smin = jnp.full((LANES,), jnp.min(srm), jnp.float32)
    s0 = jnp.full((LANES,), NEG, jnp.float32)

    def group(g, carry):
        S, smin = carry
        rm = rmb[pl.ds(g * LANES, LANES)]
        mask = rm >= smin

        def cond(c):
            m, _, _ = c
            return jnp.any(m)

        def step(c):
            m, S, smin = c
            ffsv = plsc.all_reduce_ffs(m)
            row = (g * LANES + ffsv) * T
            vs = [plsc.load_gather(buf, [row + j * LANES + lanes])
                  for j in range(NG)]
            S = _bmerge(S, _tournament(vs))
            smin = jnp.full((LANES,), jnp.min(S), jnp.float32)
            m = jnp.logical_and(m, lanes != ffsv)
            m = jnp.logical_and(m, rm >= smin)
            return m, S, smin

        _, S, smin = lax.while_loop(cond, step, (mask, S, smin))
        return S, smin

    S, _ = lax.fori_loop(0, NG, group, (s0, smin))
    return jnp.sum(S) * (1.0 / K)


def _sc_topk_body(flat, rowmax, out, buf0, buf1, rmb0, rmb1, res,
                  sem0, sem1, semr0, semr1):
    w = lax.axis_index("s") * NC + lax.axis_index("c")
    base = w * SPW
    bufs, rmbs = (buf0, buf1), (rmb0, rmb1)
    sems, semrs = (sem0, sem1), (semr0, semr1)
    zero = jnp.zeros((LANES,), jnp.float32)
    lanes = lax.broadcasted_iota(jnp.int32, (LANES,), 0)

    for b in (0, 1):
        pltpu.async_copy(flat.at[base + b], bufs[b], sems[b])
        pltpu.async_copy(rowmax.at[base + b], rmbs[b], semrs[b])

    def pair(i, carry):
        r0, r1 = carry
        for b in (0, 1):
            sl = i + b
            pltpu.make_async_copy(flat.at[base], bufs[b], sems[b]).wait()
            pltpu.make_async_copy(rowmax.at[base], rmbs[b], semrs[b]).wait()
            t = _process_slice(bufs[b], rmbs[b])

            @pl.when(sl + 2 < SPW)
            def _():
                pltpu.async_copy(flat.at[base + sl + 2], bufs[b], sems[b])
                pltpu.async_copy(rowmax.at[base + sl + 2], rmbs[b], semrs[b])

            r0 = jnp.where(lanes == sl, t, r0)
            r1 = jnp.where(lanes == sl - LANES, t, r1)
        return r0, r1

    r0, r1 = pl.loop(0, SPW, step=2, init_carry=(zero, zero))(pair)
    res[pl.ds(0, LANES)] = r0
    res[pl.ds(LANES, LANES)] = r1
    pltpu.sync_copy(res.at[pl.ds(0, SPW)], out.at[pl.ds(base, SPW)])


_sc_topk = pl.kernel(
    _sc_topk_body,
    out_type=jax.ShapeDtypeStruct((BC,), jnp.float32),
    mesh=plsc.VectorSubcoreMesh(core_axis_name="c", subcore_axis_name="s",
                                num_cores=NC, num_subcores=NS),
    scratch_types=[
        pltpu.VMEM((N,), jnp.float32),
        pltpu.VMEM((N,), jnp.float32),
        pltpu.VMEM((T,), jnp.float32),
        pltpu.VMEM((T,), jnp.float32),
        pltpu.VMEM((2 * LANES,), jnp.float32),
        pltpu.SemaphoreType.DMA,
        pltpu.SemaphoreType.DMA,
        pltpu.SemaphoreType.DMA,
        pltpu.SemaphoreType.DMA,
    ],
    compiler_params=pltpu.CompilerParams(needs_layout_passes=False),
)


def kernel(x):
    B, C, _, _ = x.shape
    xr = x.reshape(B * C, F, T)
    pooled, rowmax = _tc_pool(xr)
    flat = pooled.reshape(B * C, N)
    out = _sc_topk(flat, rowmax.reshape(B * C, T))
    return out.reshape(B, C)


# speedup vs baseline: 83.6662x; 1.8494x over previous
import jax
import jax.numpy as jnp
from jax import lax
from jax.experimental import pallas as pl
from jax.experimental.pallas import tpu as pltpu
from jax.experimental.pallas import tpu_sc as plsc

W = 8
K = 16
F = 224
T = 224
FO = F - W + 1
BC = 4 * 192
NEG = float("-inf")
GCH = 8
FP = 224
HR = BC * FP

NC = 2
NS = 16
NW = NC * NS
SPW = BC // NW
LANES = 16
NG = T // LANES


def _pool_body(x_ref, ol_ref, or_ref, rm_ref):
    a = x_ref[...]
    b = a[:, :-1, :] + a[:, 1:, :]
    b = b[:, :-2, :] + b[:, 2:, :]
    v = b[:, :-4, :] + b[:, 4:, :]
    h = v
    for d in (1, 2, 4):
        h = h + jnp.concatenate([h[:, :, d:], h[:, :, :d]], axis=2)
    col = lax.broadcasted_iota(jnp.int32, h.shape, 2)
    o = jnp.where(col < FO, h * (1.0 / (W * W)), NEG)
    rm = jnp.max(o, axis=2)
    rm_ref[...] = jnp.concatenate(
        [rm, jnp.full((GCH, FP - FO), NEG, jnp.float32)], axis=1)[:, None, :]
    opad = jnp.concatenate(
        [o, jnp.full((GCH, FP - FO, T), NEG, jnp.float32)], axis=1)
    ol_ref[...] = opad[:, :, 0:128].reshape(GCH * FP, 128)
    orr = jnp.concatenate(
        [opad[:, :, 128:T], jnp.full((GCH, FP, 256 - T), NEG, jnp.float32)],
        axis=2)
    or_ref[...] = orr.reshape(GCH * FP, 128)


def _tc_pool(xr):
    n = xr.shape[0]
    return pl.pallas_call(
        _pool_body,
        grid=(n // GCH,),
        in_specs=[pl.BlockSpec((GCH, F, T), lambda i: (i, 0, 0))],
        out_specs=[pl.BlockSpec((GCH * FP, 128), lambda i: (i, 0)),
                   pl.BlockSpec((GCH * FP, 128), lambda i: (i, 0)),
                   pl.BlockSpec((GCH, 1, T), lambda i: (i, 0, 0))],
        out_shape=[jax.ShapeDtypeStruct((n * FP, 128), jnp.float32),
                   jax.ShapeDtypeStruct((n * FP, 128), jnp.float32),
                   jax.ShapeDtypeStruct((n, 1, T), jnp.float32)],
    )(xr)


def _bmerge(a, b):
    return jnp.sort(jnp.maximum(a, jnp.flip(b)))


def _tournament(vs):
    level = [jnp.sort(v) for v in vs]
    while len(level) > 1:
        nxt = [_bmerge(level[i], level[i + 1])
               for i in range(0, len(level) - 1, 2)]
        if len(level) % 2:
            nxt.append(level[-1])
        level = nxt
    return level[0]


def _process_slice(bufl, bufr, rmb):
    lanes = lax.broadcasted_iota(jnp.int32, (LANES,), 0)
    srm = _tournament([rmb[pl.ds(g * LANES, LANES)] for g in range(NG)])
    smin = jnp.full((LANES,), jnp.min(srm), jnp.float32)
    s0 = jnp.full((LANES,), NEG, jnp.float32)

    def group(g, carry):
        S, smin = carry
        rm = rmb[pl.ds(g * LANES, LANES)]
        mask = rm >= smin

        def cond(c):
            m, _, _ = c
            return jnp.any(m)

        def step(c):
            m, S, smin = c
            ffsv = plsc.all_reduce_ffs(m)
            row = g * LANES + ffsv
            vs = [plsc.load_gather(bufl, [row, j * LANES + lanes])
                  for j in range(8)]
            vs += [plsc.load_gather(bufr, [row, j * LANES + lanes])
                   for j in range(NG - 8)]
            S = _bmerge(S, _tournament(vs))
            smin = jnp.full((LANES,), jnp.min(S), jnp.float32)
            m = jnp.logical_and(m, lanes != ffsv)
            m = jnp.logical_and(m, rm >= smin)
            return m, S, smin

        _, S, smin = lax.while_loop(cond, step, (mask, S, smin))
        return S, smin

    S, _ = lax.fori_loop(0, NG, group, (s0, smin))
    return jnp.sum(S) * (1.0 / K)


def _sc_topk_body(flatl, flatr, rowmax, out,
                  bl0, bl1, br0, br1, rmb0, rmb1, res,
                  seml0, seml1, semr0, semr1, semm0, semm1):
    w = lax.axis_index("s") * NC + lax.axis_index("c")
    base = w * SPW
    bls, brs, rmbs = (bl0, bl1), (br0, br1), (rmb0, rmb1)
    semls, semrs, semms = (seml0, seml1), (semr0, semr1), (semm0, semm1)
    zero = jnp.zeros((LANES,), jnp.float32)
    lanes = lax.broadcasted_iota(jnp.int32, (LANES,), 0)

    def start(sl, b):
        pltpu.async_copy(flatl.at[pl.ds((base + sl) * FP, FP)], bls[b], semls[b])
        pltpu.async_copy(flatr.at[pl.ds((base + sl) * FP, FP)], brs[b], semrs[b])
        pltpu.async_copy(rowmax.at[base + sl], rmbs[b], semms[b])

    for b in (0, 1):
        start(b, b)

    def pair(i, carry):
        r0, r1 = carry
        for b in (0, 1):
            sl = i + b
            pltpu.make_async_copy(flatl.at[pl.ds(0, FP)], bls[b], semls[b]).wait()
            pltpu.make_async_copy(flatr.at[pl.ds(0, FP)], brs[b], semrs[b]).wait()
            pltpu.make_async_copy(rowmax.at[base], rmbs[b], semms[b]).wait()
            t = _process_slice(bls[b], brs[b], rmbs[b])

            @pl.when(sl + 2 < SPW)
            def _():
                start(sl + 2, b)

            r0 = jnp.where(lanes == sl, t, r0)
            r1 = jnp.where(lanes == sl - LANES, t, r1)
        return r0, r1

    r0, r1 = pl.loop(0, SPW, step=2, init_carry=(zero, zero))(pair)
    res[pl.ds(0, LANES)] = r0
    res[pl.ds(LANES, LANES)] = r1
    pltpu.sync_copy(res.at[pl.ds(0, SPW)], out.at[pl.ds(base, SPW)])


_sc_topk = pl.kernel(
    _sc_topk_body,
    out_type=jax.ShapeDtypeStruct((BC,), jnp.float32),
    mesh=plsc.VectorSubcoreMesh(core_axis_name="c", subcore_axis_name="s",
                                num_cores=NC, num_subcores=NS),
    scratch_types=[
        pltpu.VMEM((FP, 128), jnp.float32),
        pltpu.VMEM((FP, 128), jnp.float32),
        pltpu.VMEM((FP, 128), jnp.float32),
        pltpu.VMEM((FP, 128), jnp.float32),
        pltpu.VMEM((T,), jnp.float32),
        pltpu.VMEM((T,), jnp.float32),
        pltpu.VMEM((2 * LANES,), jnp.float32),
        pltpu.SemaphoreType.DMA,
        pltpu.SemaphoreType.DMA,
        pltpu.SemaphoreType.DMA,
        pltpu.SemaphoreType.DMA,
        pltpu.SemaphoreType.DMA,
        pltpu.SemaphoreType.DMA,
    ],
    compiler_params=pltpu.CompilerParams(needs_layout_passes=False),
)


def kernel(x):
    B, C, _, _ = x.shape
    xr = x.reshape(B * C, F, T)
    pooledl, pooledr, rowmax = _tc_pool(xr)
    out = _sc_topk(pooledl, pooledr, rowmax.reshape(B * C, T))
    return out.reshape(B, C)
